# baseline (speedup 1.0000x reference)
import jax
import jax.numpy as jnp
from jax.experimental import pallas as pl


_N = 2097152
_BLK = 16384


def _body(tok_ref, w_ref, b_ref, x_ref, o_ref):
    wt = w_ref[...].T
    bb = b_ref[...]
    row0 = jnp.dot(tok_ref[...], wt, preferred_element_type=jnp.float32) + bb
    y1 = jnp.dot(x_ref[0], wt, preferred_element_type=jnp.float32) + bb
    o_ref[0] = jnp.broadcast_to(row0, y1.shape)
    o_ref[1] = y1


def kernel(x, token, W, b):
    tok2 = token.reshape(1, 8)
    b2 = b.reshape(1, 16)
    nblk = _N // _BLK
    out = pl.pallas_call(
        _body,
        grid=(nblk,),
        in_specs=[
            pl.BlockSpec((1, 8), lambda i: (0, 0)),
            pl.BlockSpec((16, 8), lambda i: (0, 0)),
            pl.BlockSpec((1, 16), lambda i: (0, 0)),
            pl.BlockSpec((1, _BLK, 8), lambda i: (1, i, 0)),
        ],
        out_specs=pl.BlockSpec((2, _BLK, 16), lambda i: (0, i, 0)),
        out_shape=jax.ShapeDtypeStruct((2, _N, 16), jnp.float32),
    )(tok2, W, b2, x)
    return out
